# baseline (speedup 1.0000x reference)
import functools

import numpy as np
import jax
import jax.numpy as jnp
from jax import lax
from jax.experimental import pallas as pl
from jax.experimental.pallas import tpu as pltpu
from jax.experimental.pallas import tpu_sc as plsc

_MARGIN = np.float32(0.1)
_N_SAMPLES = 1_000_000

_NW = 32
_B = 4096

_idx_cache = {}


def _rotl32(x, r):
    return ((x << np.uint32(r)) | (x >> np.uint32(32 - r))).astype(np.uint32)


def _threefry2x32(ks, x0, x1):
    x0 = x0.astype(np.uint32).copy()
    x1 = x1.astype(np.uint32).copy()
    ks0, ks1 = np.uint32(ks[0]), np.uint32(ks[1])
    ks2 = np.uint32(ks0 ^ ks1 ^ np.uint32(0x1BD11BDA))
    rot_a, rot_b = (13, 15, 26, 6), (17, 29, 16, 24)
    x0 += ks0
    x1 += ks1
    sched = [(ks1, ks2), (ks2, ks0), (ks0, ks1), (ks1, ks2), (ks2, ks0)]
    for i in range(5):
        for r in rot_a if i % 2 == 0 else rot_b:
            x0 += x1
            x1 = _rotl32(x1, r)
            x1 ^= x0
        a, b = sched[i]
        x0 += a
        x1 += b + np.uint32(i + 1)
    return x0, x1


def _np_split(key):
    b1, b2 = _threefry2x32(key, np.zeros(2, np.uint32),
                           np.arange(2, dtype=np.uint32))
    return np.stack([b1, b2], axis=1)


def _np_random_bits(key, n):
    b1, b2 = _threefry2x32(key, np.zeros(n, np.uint32),
                           np.arange(n, dtype=np.uint32))
    return b1 ^ b2


def _np_randint(key, n_draws, minval, maxval):
    k1, k2 = _np_split(key)
    y = _np_random_bits(k1, n_draws)
    z = _np_random_bits(k2, n_draws)
    span = np.uint32(np.uint64(maxval - minval) & np.uint64(0xFFFFFFFF))
    m16 = np.uint32(65536) % span
    mult = np.uint32(
        (np.uint64(m16) * np.uint64(m16)) & np.uint64(0xFFFFFFFF)) % span
    with np.errstate(over="ignore"):
        val = ((y % span) * mult + (z % span)) % span
    return (np.int64(minval) + val.astype(np.int64)).astype(np.int32)


def _pair_indices(n: int):
    if n not in _idx_cache:
        n_pairs = min(_N_SAMPLES, n * (n - 1) // 2)
        key = np.array([0, 42], np.uint32)
        ki, kj = _np_split(key)
        ii = _np_randint(ki, n_pairs, 0, n)
        ij = _np_randint(kj, n_pairs, 0, n)


        nblk = -(-n_pairs // (_NW * _B))
        nblk += nblk % 2
        tot = _NW * nblk * _B + 2 * _B

        def flat(a):
            out = ((np.arange(tot, dtype=np.int64) * 997) % n).astype(np.int32)
            out[:n_pairs] = a
            return out

        _idx_cache[n] = (flat(ii), flat(ij), nblk, n_pairs)
    return _idx_cache[n]


@functools.lru_cache(maxsize=None)
def _make_kernel(n: int, nblk: int, n_pairs: int):
    mesh = plsc.VectorSubcoreMesh(core_axis_name="c", subcore_axis_name="s")
    ndesc = _B // 128

    @functools.partial(
        pl.kernel,
        mesh=mesh,
        out_type=jax.ShapeDtypeStruct((_NW, 16), jnp.float32),
        scratch_types=[
            pltpu.VMEM((2, _B), jnp.int32),
            pltpu.VMEM((2, _B), jnp.int32),
            pltpu.VMEM((2, _B), jnp.float32),
            pltpu.VMEM((2, _B), jnp.float32),
            pltpu.VMEM((2, _B), jnp.float32),
            pltpu.VMEM((2, _B), jnp.float32),
            pltpu.VMEM((16,), jnp.float32),
            pltpu.SemaphoreType.DMA,
            pltpu.SemaphoreType.DMA,
        ],
    )
    def rank_loss_sc(ptab, ttab, hix, hjx, out,
                     vix, vjx, pi_v, ti_v, pj_v, tj_v, acc_v, sem_i, sem_g):
        wid = lax.axis_index("s") * 2 + lax.axis_index("c")
        iota = lax.iota(jnp.int32, 16)

        def stage_idx(g, p):
            base = (wid * nblk + g) * _B
            pltpu.async_copy(hix.at[pl.ds(base, _B)], vix.at[p], sem_i)
            pltpu.async_copy(hjx.at[pl.ds(base, _B)], vjx.at[p], sem_i)

        def wait_idx():
            pltpu.make_async_copy(hix.at[pl.ds(0, _B)], vix.at[0], sem_i).wait()
            pltpu.make_async_copy(hjx.at[pl.ds(0, _B)], vjx.at[0], sem_i).wait()

        def fire_gathers(p):
            for d in range(ndesc):
                s = pl.ds(d * 128, 128)
                pltpu.async_copy(ptab.at[vix.at[p].at[s]], pi_v.at[p].at[s], sem_g)
                pltpu.async_copy(ttab.at[vix.at[p].at[s]], ti_v.at[p].at[s], sem_g)
                pltpu.async_copy(ptab.at[vjx.at[p].at[s]], pj_v.at[p].at[s], sem_g)
                pltpu.async_copy(ttab.at[vjx.at[p].at[s]], tj_v.at[p].at[s], sem_g)

        def wait_gathers(p):
            dummy = ptab.at[pl.ds(0, _B)]
            for buf in (pi_v, ti_v, pj_v, tj_v):
                pltpu.make_async_copy(dummy, buf.at[p], sem_g).wait()

        def compute(g, p, acc):
            base = (wid * nblk + g) * _B

            def vec_body(v, acc):
                d = pl.ds(v * 16, 16)
                p_i = pi_v.at[p][d]
                t_i = ti_v.at[p][d]
                p_j = pj_v.at[p][d]
                t_j = tj_v.at[p][d]
                viol = jnp.maximum(
                    _MARGIN - jnp.sign(t_i - t_j) * (p_i - p_j), 0.0)
                k = base + v * 16 + iota
                return acc + jnp.where(k < n_pairs, viol, 0.0)

            return lax.fori_loop(0, _B // 16, vec_body, acc)

        stage_idx(0, 0)
        wait_idx()
        fire_gathers(0)
        stage_idx(1, 1)

        def block_pair(u, acc):
            def one(g, p, acc):
                wait_idx()
                fire_gathers(1 - p)
                wait_gathers(p)
                stage_idx(g + 2, p)
                return compute(g, p, acc)

            acc = one(2 * u, 0, acc)
            acc = one(2 * u + 1, 1, acc)
            return acc

        acc = lax.fori_loop(0, nblk // 2, block_pair,
                            jnp.zeros((16,), jnp.float32))
        wait_idx()
        wait_gathers(nblk % 2)
        acc_v[...] = acc
        pltpu.sync_copy(acc_v, out.at[wid])

    return rank_loss_sc


def kernel(delta_z_pred, delta_z_target):
    pred = delta_z_pred.reshape(-1)
    target = delta_z_target.reshape(-1)
    n = pred.shape[0]
    hix, hjx, nblk, n_pairs = _pair_indices(n)
    out = _make_kernel(n, nblk, n_pairs)(
        pred, target, jnp.asarray(hix), jnp.asarray(hjx))
    return jnp.sum(out) / jnp.float32(n_pairs)


# speedup vs baseline: 1.0241x; 1.0241x over previous
import functools

import numpy as np
import jax
import jax.numpy as jnp
from jax import lax
from jax.experimental import pallas as pl
from jax.experimental.pallas import tpu as pltpu
from jax.experimental.pallas import tpu_sc as plsc

_MARGIN = np.float32(0.1)
_N_SAMPLES = 1_000_000

_NW = 32
_B = 2048

_idx_cache = {}


def _rotl32(x, r):
    return ((x << np.uint32(r)) | (x >> np.uint32(32 - r))).astype(np.uint32)


def _threefry2x32(ks, x0, x1):
    x0 = x0.astype(np.uint32).copy()
    x1 = x1.astype(np.uint32).copy()
    ks0, ks1 = np.uint32(ks[0]), np.uint32(ks[1])
    ks2 = np.uint32(ks0 ^ ks1 ^ np.uint32(0x1BD11BDA))
    rot_a, rot_b = (13, 15, 26, 6), (17, 29, 16, 24)
    x0 += ks0
    x1 += ks1
    sched = [(ks1, ks2), (ks2, ks0), (ks0, ks1), (ks1, ks2), (ks2, ks0)]
    for i in range(5):
        for r in rot_a if i % 2 == 0 else rot_b:
            x0 += x1
            x1 = _rotl32(x1, r)
            x1 ^= x0
        a, b = sched[i]
        x0 += a
        x1 += b + np.uint32(i + 1)
    return x0, x1


def _np_split(key):
    b1, b2 = _threefry2x32(key, np.zeros(2, np.uint32),
                           np.arange(2, dtype=np.uint32))
    return np.stack([b1, b2], axis=1)


def _np_random_bits(key, n):
    b1, b2 = _threefry2x32(key, np.zeros(n, np.uint32),
                           np.arange(n, dtype=np.uint32))
    return b1 ^ b2


def _np_randint(key, n_draws, minval, maxval):
    k1, k2 = _np_split(key)
    y = _np_random_bits(k1, n_draws)
    z = _np_random_bits(k2, n_draws)
    span = np.uint32(np.uint64(maxval - minval) & np.uint64(0xFFFFFFFF))
    m16 = np.uint32(65536) % span
    mult = np.uint32(
        (np.uint64(m16) * np.uint64(m16)) & np.uint64(0xFFFFFFFF)) % span
    with np.errstate(over="ignore"):
        val = ((y % span) * mult + (z % span)) % span
    return (np.int64(minval) + val.astype(np.int64)).astype(np.int32)


def _pair_indices(n: int):
    if n not in _idx_cache:
        n_pairs = min(_N_SAMPLES, n * (n - 1) // 2)
        key = np.array([0, 42], np.uint32)
        ki, kj = _np_split(key)
        ii = _np_randint(ki, n_pairs, 0, n)
        ij = _np_randint(kj, n_pairs, 0, n)


        nblk = -(-n_pairs // (_NW * _B))
        nblk += nblk % 2
        tot = _NW * nblk * _B + 2 * _B

        def flat(a):
            out = ((np.arange(tot, dtype=np.int64) * 997) % n).astype(np.int32)
            out[:n_pairs] = a
            return out

        _idx_cache[n] = (flat(ii), flat(ij), nblk, n_pairs)
    return _idx_cache[n]


@functools.lru_cache(maxsize=None)
def _make_kernel(n: int, nblk: int, n_pairs: int):
    mesh = plsc.VectorSubcoreMesh(core_axis_name="c", subcore_axis_name="s")
    ndesc = _B // 128

    @functools.partial(
        pl.kernel,
        mesh=mesh,
        out_type=jax.ShapeDtypeStruct((_NW, 16), jnp.float32),
        scratch_types=[
            pltpu.VMEM((2, _B), jnp.int32),
            pltpu.VMEM((2, _B), jnp.int32),
            pltpu.VMEM((2, _B), jnp.float32),
            pltpu.VMEM((2, _B), jnp.float32),
            pltpu.VMEM((2, _B), jnp.float32),
            pltpu.VMEM((2, _B), jnp.float32),
            pltpu.VMEM((16,), jnp.float32),
            pltpu.SemaphoreType.DMA,
            pltpu.SemaphoreType.DMA,
        ],
    )
    def rank_loss_sc(ptab, ttab, hix, hjx, out,
                     vix, vjx, pi_v, ti_v, pj_v, tj_v, acc_v, sem_i, sem_g):
        wid = lax.axis_index("s") * 2 + lax.axis_index("c")
        iota = lax.iota(jnp.int32, 16)

        def stage_idx(g, p):
            base = (wid * nblk + g) * _B
            pltpu.async_copy(hix.at[pl.ds(base, _B)], vix.at[p], sem_i)
            pltpu.async_copy(hjx.at[pl.ds(base, _B)], vjx.at[p], sem_i)

        def wait_idx():
            pltpu.make_async_copy(hix.at[pl.ds(0, _B)], vix.at[0], sem_i).wait()
            pltpu.make_async_copy(hjx.at[pl.ds(0, _B)], vjx.at[0], sem_i).wait()

        def fire_gathers(p):
            for d in range(ndesc):
                s = pl.ds(d * 128, 128)
                pltpu.async_copy(ptab.at[vix.at[p].at[s]], pi_v.at[p].at[s], sem_g)
                pltpu.async_copy(ttab.at[vix.at[p].at[s]], ti_v.at[p].at[s], sem_g)
                pltpu.async_copy(ptab.at[vjx.at[p].at[s]], pj_v.at[p].at[s], sem_g)
                pltpu.async_copy(ttab.at[vjx.at[p].at[s]], tj_v.at[p].at[s], sem_g)

        def wait_gathers(p):
            dummy = ptab.at[pl.ds(0, _B)]
            for buf in (pi_v, ti_v, pj_v, tj_v):
                pltpu.make_async_copy(dummy, buf.at[p], sem_g).wait()

        def compute(g, p, acc):
            base = (wid * nblk + g) * _B

            def vec_body(v, acc):
                d = pl.ds(v * 16, 16)
                p_i = pi_v.at[p][d]
                t_i = ti_v.at[p][d]
                p_j = pj_v.at[p][d]
                t_j = tj_v.at[p][d]
                viol = jnp.maximum(
                    _MARGIN - jnp.sign(t_i - t_j) * (p_i - p_j), 0.0)
                k = base + v * 16 + iota
                return acc + jnp.where(k < n_pairs, viol, 0.0)

            return lax.fori_loop(0, _B // 16, vec_body, acc)

        stage_idx(0, 0)
        wait_idx()
        fire_gathers(0)
        stage_idx(1, 1)

        def block_pair(u, acc):
            def one(g, p, acc):
                wait_idx()
                fire_gathers(1 - p)
                wait_gathers(p)
                stage_idx(g + 2, p)
                return compute(g, p, acc)

            acc = one(2 * u, 0, acc)
            acc = one(2 * u + 1, 1, acc)
            return acc

        acc = lax.fori_loop(0, nblk // 2, block_pair,
                            jnp.zeros((16,), jnp.float32))
        wait_idx()
        wait_gathers(nblk % 2)
        acc_v[...] = acc
        pltpu.sync_copy(acc_v, out.at[wid])

    return rank_loss_sc


def kernel(delta_z_pred, delta_z_target):
    pred = delta_z_pred.reshape(-1)
    target = delta_z_target.reshape(-1)
    n = pred.shape[0]
    hix, hjx, nblk, n_pairs = _pair_indices(n)
    out = _make_kernel(n, nblk, n_pairs)(
        pred, target, jnp.asarray(hix), jnp.asarray(hjx))
    return jnp.sum(out) / jnp.float32(n_pairs)
